# baseline (speedup 1.0000x reference)
import functools
import math

import jax
import jax.numpy as jnp
from jax import lax
from jax.experimental import pallas as pl
from jax.experimental.pallas import tpu as pltpu
from jax.experimental.pallas import tpu_sc as plsc

VOCAB_ = 1000000
DIM_ = 32
S_ = 64
B_ = 16384

_VBLK = 512
_NBLK = -(-VOCAB_ // _VBLK)
_VP = _NBLK * 128

_NC = 2
_NS = 16
_NW = _NC * _NS
_BPW = B_ // _NW
_CHUNK = 128
_NCHUNK = _BPW // _CHUNK

_LOG_VP1 = math.log(float(VOCAB_ + 1))


def _tc_repack_body(inT_ref, out_ref):
    ident = (lax.broadcasted_iota(jnp.int32, (DIM_, DIM_), 0) ==
             lax.broadcasted_iota(jnp.int32, (DIM_, DIM_), 1)
             ).astype(jnp.float32)
    i = pl.program_id(0)
    col = lax.broadcasted_iota(jnp.int32, (DIM_, _VBLK), 1)
    x = inT_ref[...]
    x = jnp.where(i * _VBLK + col < VOCAB_, x, 0.0)
    t = lax.dot_general(x, ident, (((0,), (0,)), ((), ())),
                        preferred_element_type=jnp.float32)
    for c in range(4):
        out_ref[:, pl.ds(c * DIM_, DIM_)] = t[c * 128:(c + 1) * 128, :]


def _tc_repack(tableT):
    return pl.pallas_call(
        _tc_repack_body,
        grid=(_NBLK,),
        in_specs=[pl.BlockSpec((DIM_, _VBLK), lambda i: (0, i))],
        out_specs=pl.BlockSpec((128, 128), lambda i: (i, 0)),
        out_shape=jax.ShapeDtypeStruct((_VP, 128), jnp.float32),
    )(tableT)


def _packed_row(v):
    return lax.shift_right_logical(v, 9) * 128 + (v & 127)


def _sc_gather_rows(train_inputs, labels, emb_p, ncw_p, sampled_ids):
    mesh = plsc.VectorSubcoreMesh(core_axis_name="c", subcore_axis_name="s")
    out_type = (
        jax.ShapeDtypeStruct((B_, 128), jnp.float32),
        jax.ShapeDtypeStruct((B_, 128), jnp.float32),
        jax.ShapeDtypeStruct((S_, 128), jnp.float32),
    )

    @functools.partial(
        pl.kernel, mesh=mesh, out_type=out_type,
        scratch_types=[
            pltpu.VMEM((2, _BPW), jnp.int32),
            pltpu.VMEM((2, _NCHUNK, _CHUNK), jnp.int32),
            pltpu.VMEM((_CHUNK, 128), jnp.float32),
            pltpu.VMEM((_CHUNK, 128), jnp.float32),
            pltpu.VMEM((S_,), jnp.int32),
            pltpu.VMEM((S_,), jnp.int32),
            pltpu.VMEM((S_, 128), jnp.float32),
            pltpu.SemaphoreType.DMA,
            pltpu.SemaphoreType.DMA,
            pltpu.SemaphoreType.DMA,
        ],
    )
    def k(ti_hbm, lb_hbm, emb_hbm, ncw_hbm, sid_hbm,
          embblk_out, wblk_out, sampblk_out,
          idx_v, idx4_v, emb_v, w_v, sidx_v, sidx4_v, samp_v,
          sem1, sem2, sem3):
        wid = lax.axis_index("s") * _NC + lax.axis_index("c")
        base = wid * _BPW
        pltpu.sync_copy(ti_hbm.at[pl.ds(base, _BPW)], idx_v.at[0])
        pltpu.sync_copy(lb_hbm.at[pl.ds(base, _BPW)], idx_v.at[1])

        def shift_body(j, _):
            t = j // (_BPW // 16)
            r = j % (_BPW // 16)
            c = r // (_CHUNK // 16)
            o = r % (_CHUNK // 16)
            v = idx_v[t, pl.ds(r * 16, 16)]
            idx4_v[t, c, pl.ds(o * 16, 16)] = _packed_row(v)
            return _

        lax.fori_loop(0, 2 * (_BPW // 16), shift_body, 0, unroll=True)

        @pl.when(wid == 0)
        def _():
            pltpu.sync_copy(sid_hbm, sidx_v)

            def sshift(j, _):
                sidx4_v[pl.ds(j * 16, 16)] = _packed_row(
                    sidx_v[pl.ds(j * 16, 16)])
                return _

            lax.fori_loop(0, S_ // 16, sshift, 0, unroll=True)
            c3 = pltpu.async_copy(ncw_hbm.at[sidx4_v], samp_v, sem3)
            c3.wait()
            pltpu.sync_copy(samp_v, sampblk_out)

        for c in range(_NCHUNK):
            off = base + c * _CHUNK
            c1 = pltpu.async_copy(emb_hbm.at[idx4_v.at[0, c]], emb_v, sem1)
            c2 = pltpu.async_copy(ncw_hbm.at[idx4_v.at[1, c]], w_v, sem2)
            c1.wait()
            c2.wait()
            pltpu.sync_copy(emb_v, embblk_out.at[pl.ds(off, _CHUNK)])
            pltpu.sync_copy(w_v, wblk_out.at[pl.ds(off, _CHUNK)])

    return k(train_inputs, labels, emb_p, ncw_p, sampled_ids)


def _sc_gather_biases(labels, nce_biases, sampled_ids):
    mesh = plsc.VectorSubcoreMesh(core_axis_name="c", subcore_axis_name="s")
    out_type = (
        jax.ShapeDtypeStruct((B_,), jnp.float32),
        jax.ShapeDtypeStruct((S_,), jnp.float32),
    )

    @functools.partial(
        pl.kernel, mesh=mesh, out_type=out_type,
        compiler_params=pltpu.CompilerParams(use_tc_tiling_on_sc=False),
        scratch_types=[
            pltpu.VMEM((_BPW,), jnp.int32),
            pltpu.VMEM((_BPW,), jnp.float32),
            pltpu.VMEM((S_,), jnp.int32),
            pltpu.VMEM((S_,), jnp.float32),
            pltpu.SemaphoreType.DMA,
            pltpu.SemaphoreType.DMA,
        ],
    )
    def k(lb_hbm, ncb_hbm, sid_hbm, trueb_out, sampb_out,
          idx_v, b_v, sidx_v, sb_v, sem1, sem2):
        wid = lax.axis_index("s") * _NC + lax.axis_index("c")
        base = wid * _BPW
        pltpu.sync_copy(lb_hbm.at[pl.ds(base, _BPW)], idx_v)
        c1 = pltpu.async_copy(ncb_hbm.at[idx_v], b_v, sem1)

        @pl.when(wid == 0)
        def _():
            pltpu.sync_copy(sid_hbm, sidx_v)
            c2 = pltpu.async_copy(ncb_hbm.at[sidx_v], sb_v, sem2)
            c2.wait()
            pltpu.sync_copy(sb_v, sampb_out)

        c1.wait()
        pltpu.sync_copy(b_v, trueb_out.at[pl.ds(base, _BPW)])

    return k(labels, nce_biases, sampled_ids)


_BB = 2048


def _select32(blk, ids):
    sel = lax.shift_right_logical(ids, 7) & 3
    out = jnp.zeros((blk.shape[0], DIM_), jnp.float32)
    for c in range(4):
        m = (sel == c).astype(jnp.float32)[:, None]
        out = out + m * blk[:, c * DIM_:(c + 1) * DIM_]
    return out


def _tc_loss_body(embblk_ref, wblk_ref, ti_ref, lb_ref, tb_ref,
                  sampblk_ref, sb_ref, sid_ref, embed_ref, out_ref):
    i = pl.program_id(0)
    ti = ti_ref[...]
    lb = lb_ref[...]
    sid = sid_ref[...]
    e = _select32(embblk_ref[...], ti)
    w = _select32(wblk_ref[...], lb)
    sw = _select32(sampblk_ref[...], sid)
    embed_ref[...] = e
    tl = jnp.sum(e * w, axis=1) + tb_ref[...]
    lf = lb.astype(jnp.float32)
    p_true = (jnp.log(lf + 2.0) - jnp.log(lf + 1.0)) / _LOG_VP1
    tl = tl - jnp.log(S_ * p_true)
    sl = lax.dot_general(e, sw, (((1,), (1,)), ((), ())),
                         preferred_element_type=jnp.float32)
    sf = sid.astype(jnp.float32)
    p_s = (jnp.log(sf + 2.0) - jnp.log(sf + 1.0)) / _LOG_VP1
    sl = sl + (sb_ref[...] - jnp.log(S_ * p_s))[None, :]
    ce_t = jnp.maximum(tl, 0.0) - tl + jnp.log1p(jnp.exp(-jnp.abs(tl)))
    ce_s = jnp.maximum(sl, 0.0) + jnp.log1p(jnp.exp(-jnp.abs(sl)))
    part = (jnp.sum(ce_t) + jnp.sum(ce_s)) * (1.0 / B_)

    @pl.when(i == 0)
    def _():
        out_ref[0, 0] = 0.0

    out_ref[0, 0] += part


def _tc_loss(embblk, wblk, train_inputs, labels, true_b, sampblk, sampled_b,
             sampled_ids, interpret=False):
    nblk = B_ // _BB
    embed, cost = pl.pallas_call(
        _tc_loss_body,
        grid=(nblk,),
        in_specs=[
            pl.BlockSpec((_BB, 128), lambda i: (i, 0)),
            pl.BlockSpec((_BB, 128), lambda i: (i, 0)),
            pl.BlockSpec((_BB,), lambda i: (i,)),
            pl.BlockSpec((_BB,), lambda i: (i,)),
            pl.BlockSpec((_BB,), lambda i: (i,)),
            pl.BlockSpec((S_, 128), lambda i: (0, 0)),
            pl.BlockSpec((S_,), lambda i: (0,)),
            pl.BlockSpec((S_,), lambda i: (0,)),
        ],
        out_specs=(
            pl.BlockSpec((_BB, DIM_), lambda i: (i, 0)),
            pl.BlockSpec((1, 1), lambda i: (0, 0), memory_space=pltpu.SMEM),
        ),
        out_shape=(
            jax.ShapeDtypeStruct((B_, DIM_), jnp.float32),
            jax.ShapeDtypeStruct((1, 1), jnp.float32),
        ),
        interpret=interpret,
    )(embblk, wblk, train_inputs, labels, true_b, sampblk, sampled_b,
      sampled_ids)
    return embed, cost[0, 0]


def kernel(train_inputs, train_labels, embeddings, nce_weights, nce_biases,
           sampled_ids):
    labels = train_labels.reshape(-1)
    emb_p = _tc_repack(embeddings.T)
    ncw_p = _tc_repack(nce_weights.T)
    embblk, wblk, sampblk = _sc_gather_rows(
        train_inputs, labels, emb_p, ncw_p, sampled_ids)
    true_b, sampled_b = _sc_gather_biases(labels, nce_biases, sampled_ids)
    embed, nce_cost = _tc_loss(embblk, wblk, train_inputs, labels, true_b,
                               sampblk, sampled_b, sampled_ids)
    return embed, nce_cost


# speedup vs baseline: 2.7973x; 2.7973x over previous
import functools
import math

import jax
import jax.numpy as jnp
from jax import lax
from jax.experimental import pallas as pl
from jax.experimental.pallas import tpu as pltpu
from jax.experimental.pallas import tpu_sc as plsc

VOCAB_ = 1000000
DIM_ = 32
S_ = 64
B_ = 16384

_VBLK = 2048
_NBLK = -(-VOCAB_ // _VBLK)
_VP = _NBLK * (_VBLK // 4)

_NC = 2
_NS = 16
_NW = _NC * _NS
_BPW = B_ // _NW
_CHUNK = 128
_NCHUNK = _BPW // _CHUNK

_LOG_VP1 = math.log(float(VOCAB_ + 1))


def _tc_repack_body(inT_ref, out_ref):
    ident = (lax.broadcasted_iota(jnp.int32, (DIM_, DIM_), 0) ==
             lax.broadcasted_iota(jnp.int32, (DIM_, DIM_), 1)
             ).astype(jnp.float32)
    i = pl.program_id(0)
    col = lax.broadcasted_iota(jnp.int32, (DIM_, _VBLK), 1)
    x = inT_ref[...]
    x = jnp.where(i * _VBLK + col < VOCAB_, x, 0.0)
    t = lax.dot_general(x, ident, (((0,), (0,)), ((), ())),
                        preferred_element_type=jnp.float32)
    for h in range(_VBLK // 512):
        for c in range(4):
            out_ref[pl.ds(128 * h, 128), pl.ds(c * DIM_, DIM_)] = (
                t[512 * h + 128 * c:512 * h + 128 * (c + 1), :])


def _tc_repack(tableT):
    return pl.pallas_call(
        _tc_repack_body,
        grid=(_NBLK,),
        in_specs=[pl.BlockSpec((DIM_, _VBLK), lambda i: (0, i))],
        out_specs=pl.BlockSpec((_VBLK // 4, 128), lambda i: (i, 0)),
        out_shape=jax.ShapeDtypeStruct((_VP, 128), jnp.float32),
    )(tableT)


def _packed_row(v):
    return lax.shift_right_logical(v, 9) * 128 + (v & 127)


def _sc_gather_rows(train_inputs, labels, emb_p, ncw_p, sampled_ids):
    mesh = plsc.VectorSubcoreMesh(core_axis_name="c", subcore_axis_name="s")
    out_type = (
        jax.ShapeDtypeStruct((B_, 128), jnp.float32),
        jax.ShapeDtypeStruct((B_, 128), jnp.float32),
        jax.ShapeDtypeStruct((S_, 128), jnp.float32),
    )

    @functools.partial(
        pl.kernel, mesh=mesh, out_type=out_type,
        scratch_types=[
            pltpu.VMEM((2, _BPW), jnp.int32),
            pltpu.VMEM((2, _NCHUNK, _CHUNK), jnp.int32),
            pltpu.VMEM((_CHUNK, 128), jnp.float32),
            pltpu.VMEM((_CHUNK, 128), jnp.float32),
            pltpu.VMEM((S_,), jnp.int32),
            pltpu.VMEM((S_,), jnp.int32),
            pltpu.VMEM((S_, 128), jnp.float32),
            pltpu.SemaphoreType.DMA,
            pltpu.SemaphoreType.DMA,
            pltpu.SemaphoreType.DMA,
        ],
    )
    def k(ti_hbm, lb_hbm, emb_hbm, ncw_hbm, sid_hbm,
          embblk_out, wblk_out, sampblk_out,
          idx_v, idx4_v, emb_v, w_v, sidx_v, sidx4_v, samp_v,
          sem1, sem2, sem3):
        wid = lax.axis_index("s") * _NC + lax.axis_index("c")
        base = wid * _BPW
        pltpu.sync_copy(ti_hbm.at[pl.ds(base, _BPW)], idx_v.at[0])
        pltpu.sync_copy(lb_hbm.at[pl.ds(base, _BPW)], idx_v.at[1])

        def shift_body(j, _):
            t = j // (_BPW // 16)
            r = j % (_BPW // 16)
            c = r // (_CHUNK // 16)
            o = r % (_CHUNK // 16)
            v = idx_v[t, pl.ds(r * 16, 16)]
            idx4_v[t, c, pl.ds(o * 16, 16)] = _packed_row(v)
            return _

        lax.fori_loop(0, 2 * (_BPW // 16), shift_body, 0, unroll=True)

        @pl.when(wid == 0)
        def _():
            pltpu.sync_copy(sid_hbm, sidx_v)

            def sshift(j, _):
                sidx4_v[pl.ds(j * 16, 16)] = _packed_row(
                    sidx_v[pl.ds(j * 16, 16)])
                return _

            lax.fori_loop(0, S_ // 16, sshift, 0, unroll=True)
            c3 = pltpu.async_copy(ncw_hbm.at[sidx4_v], samp_v, sem3)
            c3.wait()
            pltpu.sync_copy(samp_v, sampblk_out)

        for c in range(_NCHUNK):
            off = base + c * _CHUNK
            c1 = pltpu.async_copy(emb_hbm.at[idx4_v.at[0, c]], emb_v, sem1)
            c2 = pltpu.async_copy(ncw_hbm.at[idx4_v.at[1, c]], w_v, sem2)
            c1.wait()
            c2.wait()
            pltpu.sync_copy(emb_v, embblk_out.at[pl.ds(off, _CHUNK)])
            pltpu.sync_copy(w_v, wblk_out.at[pl.ds(off, _CHUNK)])

    return k(train_inputs, labels, emb_p, ncw_p, sampled_ids)


def _sc_gather_biases(labels, nce_biases, sampled_ids):
    mesh = plsc.VectorSubcoreMesh(core_axis_name="c", subcore_axis_name="s")
    out_type = (
        jax.ShapeDtypeStruct((B_,), jnp.float32),
        jax.ShapeDtypeStruct((S_,), jnp.float32),
    )

    @functools.partial(
        pl.kernel, mesh=mesh, out_type=out_type,
        compiler_params=pltpu.CompilerParams(use_tc_tiling_on_sc=False),
        scratch_types=[
            pltpu.VMEM((_BPW,), jnp.int32),
            pltpu.VMEM((_BPW,), jnp.float32),
            pltpu.VMEM((S_,), jnp.int32),
            pltpu.VMEM((S_,), jnp.float32),
            pltpu.SemaphoreType.DMA,
            pltpu.SemaphoreType.DMA,
        ],
    )
    def k(lb_hbm, ncb_hbm, sid_hbm, trueb_out, sampb_out,
          idx_v, b_v, sidx_v, sb_v, sem1, sem2):
        wid = lax.axis_index("s") * _NC + lax.axis_index("c")
        base = wid * _BPW
        pltpu.sync_copy(lb_hbm.at[pl.ds(base, _BPW)], idx_v)
        c1 = pltpu.async_copy(ncb_hbm.at[idx_v], b_v, sem1)

        @pl.when(wid == 0)
        def _():
            pltpu.sync_copy(sid_hbm, sidx_v)
            c2 = pltpu.async_copy(ncb_hbm.at[sidx_v], sb_v, sem2)
            c2.wait()
            pltpu.sync_copy(sb_v, sampb_out)

        c1.wait()
        pltpu.sync_copy(b_v, trueb_out.at[pl.ds(base, _BPW)])

    return k(labels, nce_biases, sampled_ids)


_BB = 2048


def _select32(blk, ids):
    sel = lax.shift_right_logical(ids, 7) & 3
    out = jnp.zeros((blk.shape[0], DIM_), jnp.float32)
    for c in range(4):
        m = (sel == c).astype(jnp.float32)[:, None]
        out = out + m * blk[:, c * DIM_:(c + 1) * DIM_]
    return out


def _tc_loss_body(embblk_ref, wblk_ref, ti_ref, lb_ref, tb_ref,
                  sampblk_ref, sb_ref, sid_ref, embed_ref, out_ref):
    i = pl.program_id(0)
    ti = ti_ref[...]
    lb = lb_ref[...]
    sid = sid_ref[...]
    e = _select32(embblk_ref[...], ti)
    w = _select32(wblk_ref[...], lb)
    sw = _select32(sampblk_ref[...], sid)
    embed_ref[...] = e
    tl = jnp.sum(e * w, axis=1) + tb_ref[...]
    lf = lb.astype(jnp.float32)
    p_true = (jnp.log(lf + 2.0) - jnp.log(lf + 1.0)) / _LOG_VP1
    tl = tl - jnp.log(S_ * p_true)
    sl = lax.dot_general(e, sw, (((1,), (1,)), ((), ())),
                         preferred_element_type=jnp.float32)
    sf = sid.astype(jnp.float32)
    p_s = (jnp.log(sf + 2.0) - jnp.log(sf + 1.0)) / _LOG_VP1
    sl = sl + (sb_ref[...] - jnp.log(S_ * p_s))[None, :]
    ce_t = jnp.maximum(tl, 0.0) - tl + jnp.log1p(jnp.exp(-jnp.abs(tl)))
    ce_s = jnp.maximum(sl, 0.0) + jnp.log1p(jnp.exp(-jnp.abs(sl)))
    part = (jnp.sum(ce_t) + jnp.sum(ce_s)) * (1.0 / B_)

    @pl.when(i == 0)
    def _():
        out_ref[0, 0] = 0.0

    out_ref[0, 0] += part


def _tc_loss(embblk, wblk, train_inputs, labels, true_b, sampblk, sampled_b,
             sampled_ids, interpret=False):
    nblk = B_ // _BB
    embed, cost = pl.pallas_call(
        _tc_loss_body,
        grid=(nblk,),
        in_specs=[
            pl.BlockSpec((_BB, 128), lambda i: (i, 0)),
            pl.BlockSpec((_BB, 128), lambda i: (i, 0)),
            pl.BlockSpec((_BB,), lambda i: (i,)),
            pl.BlockSpec((_BB,), lambda i: (i,)),
            pl.BlockSpec((_BB,), lambda i: (i,)),
            pl.BlockSpec((S_, 128), lambda i: (0, 0)),
            pl.BlockSpec((S_,), lambda i: (0,)),
            pl.BlockSpec((S_,), lambda i: (0,)),
        ],
        out_specs=(
            pl.BlockSpec((_BB, DIM_), lambda i: (i, 0)),
            pl.BlockSpec((1, 1), lambda i: (0, 0), memory_space=pltpu.SMEM),
        ),
        out_shape=(
            jax.ShapeDtypeStruct((B_, DIM_), jnp.float32),
            jax.ShapeDtypeStruct((1, 1), jnp.float32),
        ),
        interpret=interpret,
    )(embblk, wblk, train_inputs, labels, true_b, sampblk, sampled_b,
      sampled_ids)
    return embed, cost[0, 0]


def kernel(train_inputs, train_labels, embeddings, nce_weights, nce_biases,
           sampled_ids):
    labels = train_labels.reshape(-1)
    emb_p = _tc_repack(embeddings.T)
    ncw_p = _tc_repack(nce_weights.T)
    embblk, wblk, sampblk = _sc_gather_rows(
        train_inputs, labels, emb_p, ncw_p, sampled_ids)
    true_b, sampled_b = _sc_gather_biases(labels, nce_biases, sampled_ids)
    embed, nce_cost = _tc_loss(embblk, wblk, train_inputs, labels, true_b,
                               sampblk, sampled_b, sampled_ids)
    return embed, nce_cost


# speedup vs baseline: 4.3153x; 1.5426x over previous
import functools
import math

import jax
import jax.numpy as jnp
from jax import lax
from jax.experimental import pallas as pl
from jax.experimental.pallas import tpu as pltpu
from jax.experimental.pallas import tpu_sc as plsc

VOCAB_ = 1000000
DIM_ = 32
S_ = 64
B_ = 16384

_VBLK = 8192
_NBLK = -(-VOCAB_ // _VBLK)
_VP = _NBLK * (_VBLK // 4)

_NC = 2
_NS = 16
_NW = _NC * _NS
_BPW = B_ // _NW
_CHUNK = 128
_NCHUNK = _BPW // _CHUNK

_LOG_VP1 = math.log(float(VOCAB_ + 1))


def _tc_repack_body(inT_ref, out_ref):
    ident = (lax.broadcasted_iota(jnp.int32, (DIM_, DIM_), 0) ==
             lax.broadcasted_iota(jnp.int32, (DIM_, DIM_), 1)
             ).astype(jnp.float32)
    i = pl.program_id(0)
    col = lax.broadcasted_iota(jnp.int32, (DIM_, _VBLK), 1)
    x = inT_ref[...]
    x = jnp.where(i * _VBLK + col < VOCAB_, x, 0.0)
    t = lax.dot_general(x, ident, (((0,), (0,)), ((), ())),
                        preferred_element_type=jnp.float32)
    for h in range(_VBLK // 512):
        for c in range(4):
            out_ref[pl.ds(128 * h, 128), pl.ds(c * DIM_, DIM_)] = (
                t[512 * h + 128 * c:512 * h + 128 * (c + 1), :])


def _tc_repack(tableT):
    return pl.pallas_call(
        _tc_repack_body,
        grid=(_NBLK,),
        in_specs=[pl.BlockSpec((DIM_, _VBLK), lambda i: (0, i))],
        out_specs=pl.BlockSpec((_VBLK // 4, 128), lambda i: (i, 0)),
        out_shape=jax.ShapeDtypeStruct((_VP, 128), jnp.float32),
    )(tableT)


def _packed_row(v):
    return lax.shift_right_logical(v, 9) * 128 + (v & 127)


def _sc_gather_rows(train_inputs, labels, emb_p, ncw_p, sampled_ids):
    mesh = plsc.VectorSubcoreMesh(core_axis_name="c", subcore_axis_name="s")
    out_type = (
        jax.ShapeDtypeStruct((B_, 128), jnp.float32),
        jax.ShapeDtypeStruct((B_, 128), jnp.float32),
        jax.ShapeDtypeStruct((S_, 128), jnp.float32),
    )

    @functools.partial(
        pl.kernel, mesh=mesh, out_type=out_type,
        scratch_types=[
            pltpu.VMEM((2, _BPW), jnp.int32),
            pltpu.VMEM((2, _NCHUNK, _CHUNK), jnp.int32),
            pltpu.VMEM((_CHUNK, 128), jnp.float32),
            pltpu.VMEM((_CHUNK, 128), jnp.float32),
            pltpu.VMEM((S_,), jnp.int32),
            pltpu.VMEM((S_,), jnp.int32),
            pltpu.VMEM((S_, 128), jnp.float32),
            pltpu.SemaphoreType.DMA,
            pltpu.SemaphoreType.DMA,
            pltpu.SemaphoreType.DMA,
        ],
    )
    def k(ti_hbm, lb_hbm, emb_hbm, ncw_hbm, sid_hbm,
          embblk_out, wblk_out, sampblk_out,
          idx_v, idx4_v, emb_v, w_v, sidx_v, sidx4_v, samp_v,
          sem1, sem2, sem3):
        wid = lax.axis_index("s") * _NC + lax.axis_index("c")
        base = wid * _BPW
        pltpu.sync_copy(ti_hbm.at[pl.ds(base, _BPW)], idx_v.at[0])
        pltpu.sync_copy(lb_hbm.at[pl.ds(base, _BPW)], idx_v.at[1])

        def shift_body(j, _):
            t = j // (_BPW // 16)
            r = j % (_BPW // 16)
            c = r // (_CHUNK // 16)
            o = r % (_CHUNK // 16)
            v = idx_v[t, pl.ds(r * 16, 16)]
            idx4_v[t, c, pl.ds(o * 16, 16)] = _packed_row(v)
            return _

        lax.fori_loop(0, 2 * (_BPW // 16), shift_body, 0, unroll=True)

        @pl.when(wid == 0)
        def _():
            pltpu.sync_copy(sid_hbm, sidx_v)

            def sshift(j, _):
                sidx4_v[pl.ds(j * 16, 16)] = _packed_row(
                    sidx_v[pl.ds(j * 16, 16)])
                return _

            lax.fori_loop(0, S_ // 16, sshift, 0, unroll=True)
            c3 = pltpu.async_copy(ncw_hbm.at[sidx4_v], samp_v, sem3)
            c3.wait()
            pltpu.sync_copy(samp_v, sampblk_out)

        for c in range(_NCHUNK):
            off = base + c * _CHUNK
            c1 = pltpu.async_copy(emb_hbm.at[idx4_v.at[0, c]], emb_v, sem1)
            c2 = pltpu.async_copy(ncw_hbm.at[idx4_v.at[1, c]], w_v, sem2)
            c1.wait()
            c2.wait()
            pltpu.sync_copy(emb_v, embblk_out.at[pl.ds(off, _CHUNK)])
            pltpu.sync_copy(w_v, wblk_out.at[pl.ds(off, _CHUNK)])

    return k(train_inputs, labels, emb_p, ncw_p, sampled_ids)


def _sc_gather_biases(labels, nce_biases, sampled_ids):
    mesh = plsc.VectorSubcoreMesh(core_axis_name="c", subcore_axis_name="s")
    out_type = (
        jax.ShapeDtypeStruct((B_,), jnp.float32),
        jax.ShapeDtypeStruct((S_,), jnp.float32),
    )

    @functools.partial(
        pl.kernel, mesh=mesh, out_type=out_type,
        compiler_params=pltpu.CompilerParams(use_tc_tiling_on_sc=False),
        scratch_types=[
            pltpu.VMEM((_BPW,), jnp.int32),
            pltpu.VMEM((_BPW,), jnp.float32),
            pltpu.VMEM((S_,), jnp.int32),
            pltpu.VMEM((S_,), jnp.float32),
            pltpu.SemaphoreType.DMA,
            pltpu.SemaphoreType.DMA,
        ],
    )
    def k(lb_hbm, ncb_hbm, sid_hbm, trueb_out, sampb_out,
          idx_v, b_v, sidx_v, sb_v, sem1, sem2):
        wid = lax.axis_index("s") * _NC + lax.axis_index("c")
        base = wid * _BPW
        pltpu.sync_copy(lb_hbm.at[pl.ds(base, _BPW)], idx_v)
        c1 = pltpu.async_copy(ncb_hbm.at[idx_v], b_v, sem1)

        @pl.when(wid == 0)
        def _():
            pltpu.sync_copy(sid_hbm, sidx_v)
            c2 = pltpu.async_copy(ncb_hbm.at[sidx_v], sb_v, sem2)
            c2.wait()
            pltpu.sync_copy(sb_v, sampb_out)

        c1.wait()
        pltpu.sync_copy(b_v, trueb_out.at[pl.ds(base, _BPW)])

    return k(labels, nce_biases, sampled_ids)


_BB = 2048


def _select32(blk, ids):
    sel = lax.shift_right_logical(ids, 7) & 3
    out = jnp.zeros((blk.shape[0], DIM_), jnp.float32)
    for c in range(4):
        m = (sel == c).astype(jnp.float32)[:, None]
        out = out + m * blk[:, c * DIM_:(c + 1) * DIM_]
    return out


def _tc_loss_body(embblk_ref, wblk_ref, ti_ref, lb_ref, tb_ref,
                  sampblk_ref, sb_ref, sid_ref, embed_ref, out_ref):
    i = pl.program_id(0)
    ti = ti_ref[...]
    lb = lb_ref[...]
    sid = sid_ref[...]
    e = _select32(embblk_ref[...], ti)
    w = _select32(wblk_ref[...], lb)
    sw = _select32(sampblk_ref[...], sid)
    embed_ref[...] = e
    tl = jnp.sum(e * w, axis=1) + tb_ref[...]
    lf = lb.astype(jnp.float32)
    p_true = (jnp.log(lf + 2.0) - jnp.log(lf + 1.0)) / _LOG_VP1
    tl = tl - jnp.log(S_ * p_true)
    sl = lax.dot_general(e, sw, (((1,), (1,)), ((), ())),
                         preferred_element_type=jnp.float32)
    sf = sid.astype(jnp.float32)
    p_s = (jnp.log(sf + 2.0) - jnp.log(sf + 1.0)) / _LOG_VP1
    sl = sl + (sb_ref[...] - jnp.log(S_ * p_s))[None, :]
    ce_t = jnp.maximum(tl, 0.0) - tl + jnp.log1p(jnp.exp(-jnp.abs(tl)))
    ce_s = jnp.maximum(sl, 0.0) + jnp.log1p(jnp.exp(-jnp.abs(sl)))
    part = (jnp.sum(ce_t) + jnp.sum(ce_s)) * (1.0 / B_)

    @pl.when(i == 0)
    def _():
        out_ref[0, 0] = 0.0

    out_ref[0, 0] += part


def _tc_loss(embblk, wblk, train_inputs, labels, true_b, sampblk, sampled_b,
             sampled_ids, interpret=False):
    nblk = B_ // _BB
    embed, cost = pl.pallas_call(
        _tc_loss_body,
        grid=(nblk,),
        in_specs=[
            pl.BlockSpec((_BB, 128), lambda i: (i, 0)),
            pl.BlockSpec((_BB, 128), lambda i: (i, 0)),
            pl.BlockSpec((_BB,), lambda i: (i,)),
            pl.BlockSpec((_BB,), lambda i: (i,)),
            pl.BlockSpec((_BB,), lambda i: (i,)),
            pl.BlockSpec((S_, 128), lambda i: (0, 0)),
            pl.BlockSpec((S_,), lambda i: (0,)),
            pl.BlockSpec((S_,), lambda i: (0,)),
        ],
        out_specs=(
            pl.BlockSpec((_BB, DIM_), lambda i: (i, 0)),
            pl.BlockSpec((1, 1), lambda i: (0, 0), memory_space=pltpu.SMEM),
        ),
        out_shape=(
            jax.ShapeDtypeStruct((B_, DIM_), jnp.float32),
            jax.ShapeDtypeStruct((1, 1), jnp.float32),
        ),
        interpret=interpret,
    )(embblk, wblk, train_inputs, labels, true_b, sampblk, sampled_b,
      sampled_ids)
    return embed, cost[0, 0]


def kernel(train_inputs, train_labels, embeddings, nce_weights, nce_biases,
           sampled_ids):
    labels = train_labels.reshape(-1)
    emb_p = _tc_repack(embeddings.T)
    ncw_p = _tc_repack(nce_weights.T)
    embblk, wblk, sampblk = _sc_gather_rows(
        train_inputs, labels, emb_p, ncw_p, sampled_ids)
    true_b, sampled_b = _sc_gather_biases(labels, nce_biases, sampled_ids)
    embed, nce_cost = _tc_loss(embblk, wblk, train_inputs, labels, true_b,
                               sampblk, sampled_b, sampled_ids)
    return embed, nce_cost


# speedup vs baseline: 4.4178x; 1.0238x over previous
import functools
import math

import jax
import jax.numpy as jnp
from jax import lax
from jax.experimental import pallas as pl
from jax.experimental.pallas import tpu as pltpu
from jax.experimental.pallas import tpu_sc as plsc

VOCAB_ = 1000000
DIM_ = 32
S_ = 64
B_ = 16384

_VBLK = 32768
_NBLK = -(-VOCAB_ // _VBLK)
_VP = _NBLK * (_VBLK // 4)

_NC = 2
_NS = 16
_NW = _NC * _NS
_BPW = B_ // _NW
_CHUNK = 128
_NCHUNK = _BPW // _CHUNK

_LOG_VP1 = math.log(float(VOCAB_ + 1))


def _tc_repack_body(inT_ref, out_ref):
    ident = (lax.broadcasted_iota(jnp.int32, (DIM_, DIM_), 0) ==
             lax.broadcasted_iota(jnp.int32, (DIM_, DIM_), 1)
             ).astype(jnp.float32)
    i = pl.program_id(0)
    col = lax.broadcasted_iota(jnp.int32, (DIM_, _VBLK), 1)
    x = inT_ref[...]
    x = jnp.where(i * _VBLK + col < VOCAB_, x, 0.0)
    t = lax.dot_general(x, ident, (((0,), (0,)), ((), ())),
                        preferred_element_type=jnp.float32)
    for h in range(_VBLK // 512):
        for c in range(4):
            out_ref[pl.ds(128 * h, 128), pl.ds(c * DIM_, DIM_)] = (
                t[512 * h + 128 * c:512 * h + 128 * (c + 1), :])


def _tc_repack(tableT):
    return pl.pallas_call(
        _tc_repack_body,
        grid=(_NBLK,),
        in_specs=[pl.BlockSpec((DIM_, _VBLK), lambda i: (0, i))],
        out_specs=pl.BlockSpec((_VBLK // 4, 128), lambda i: (i, 0)),
        out_shape=jax.ShapeDtypeStruct((_VP, 128), jnp.float32),
    )(tableT)


def _packed_row(v):
    return lax.shift_right_logical(v, 9) * 128 + (v & 127)


def _sc_gather_rows(train_inputs, labels, emb_p, ncw_p, sampled_ids):
    mesh = plsc.VectorSubcoreMesh(core_axis_name="c", subcore_axis_name="s")
    out_type = (
        jax.ShapeDtypeStruct((B_, 128), jnp.float32),
        jax.ShapeDtypeStruct((B_, 128), jnp.float32),
        jax.ShapeDtypeStruct((S_, 128), jnp.float32),
    )

    @functools.partial(
        pl.kernel, mesh=mesh, out_type=out_type,
        scratch_types=[
            pltpu.VMEM((2, _BPW), jnp.int32),
            pltpu.VMEM((2, _NCHUNK, _CHUNK), jnp.int32),
            pltpu.VMEM((_CHUNK, 128), jnp.float32),
            pltpu.VMEM((_CHUNK, 128), jnp.float32),
            pltpu.VMEM((S_,), jnp.int32),
            pltpu.VMEM((S_,), jnp.int32),
            pltpu.VMEM((S_, 128), jnp.float32),
            pltpu.SemaphoreType.DMA,
            pltpu.SemaphoreType.DMA,
            pltpu.SemaphoreType.DMA,
        ],
    )
    def k(ti_hbm, lb_hbm, emb_hbm, ncw_hbm, sid_hbm,
          embblk_out, wblk_out, sampblk_out,
          idx_v, idx4_v, emb_v, w_v, sidx_v, sidx4_v, samp_v,
          sem1, sem2, sem3):
        wid = lax.axis_index("s") * _NC + lax.axis_index("c")
        base = wid * _BPW
        pltpu.sync_copy(ti_hbm.at[pl.ds(base, _BPW)], idx_v.at[0])
        pltpu.sync_copy(lb_hbm.at[pl.ds(base, _BPW)], idx_v.at[1])

        def shift_body(j, _):
            t = j // (_BPW // 16)
            r = j % (_BPW // 16)
            c = r // (_CHUNK // 16)
            o = r % (_CHUNK // 16)
            v = idx_v[t, pl.ds(r * 16, 16)]
            idx4_v[t, c, pl.ds(o * 16, 16)] = _packed_row(v)
            return _

        lax.fori_loop(0, 2 * (_BPW // 16), shift_body, 0, unroll=True)

        @pl.when(wid == 0)
        def _():
            pltpu.sync_copy(sid_hbm, sidx_v)

            def sshift(j, _):
                sidx4_v[pl.ds(j * 16, 16)] = _packed_row(
                    sidx_v[pl.ds(j * 16, 16)])
                return _

            lax.fori_loop(0, S_ // 16, sshift, 0, unroll=True)
            c3 = pltpu.async_copy(ncw_hbm.at[sidx4_v], samp_v, sem3)
            c3.wait()
            pltpu.sync_copy(samp_v, sampblk_out)

        for c in range(_NCHUNK):
            off = base + c * _CHUNK
            c1 = pltpu.async_copy(emb_hbm.at[idx4_v.at[0, c]], emb_v, sem1)
            c2 = pltpu.async_copy(ncw_hbm.at[idx4_v.at[1, c]], w_v, sem2)
            c1.wait()
            c2.wait()
            pltpu.sync_copy(emb_v, embblk_out.at[pl.ds(off, _CHUNK)])
            pltpu.sync_copy(w_v, wblk_out.at[pl.ds(off, _CHUNK)])

    return k(train_inputs, labels, emb_p, ncw_p, sampled_ids)


def _sc_gather_biases(labels, nce_biases, sampled_ids):
    mesh = plsc.VectorSubcoreMesh(core_axis_name="c", subcore_axis_name="s")
    out_type = (
        jax.ShapeDtypeStruct((B_,), jnp.float32),
        jax.ShapeDtypeStruct((S_,), jnp.float32),
    )

    @functools.partial(
        pl.kernel, mesh=mesh, out_type=out_type,
        compiler_params=pltpu.CompilerParams(use_tc_tiling_on_sc=False),
        scratch_types=[
            pltpu.VMEM((_BPW,), jnp.int32),
            pltpu.VMEM((_BPW,), jnp.float32),
            pltpu.VMEM((S_,), jnp.int32),
            pltpu.VMEM((S_,), jnp.float32),
            pltpu.SemaphoreType.DMA,
            pltpu.SemaphoreType.DMA,
        ],
    )
    def k(lb_hbm, ncb_hbm, sid_hbm, trueb_out, sampb_out,
          idx_v, b_v, sidx_v, sb_v, sem1, sem2):
        wid = lax.axis_index("s") * _NC + lax.axis_index("c")
        base = wid * _BPW
        pltpu.sync_copy(lb_hbm.at[pl.ds(base, _BPW)], idx_v)
        c1 = pltpu.async_copy(ncb_hbm.at[idx_v], b_v, sem1)

        @pl.when(wid == 0)
        def _():
            pltpu.sync_copy(sid_hbm, sidx_v)
            c2 = pltpu.async_copy(ncb_hbm.at[sidx_v], sb_v, sem2)
            c2.wait()
            pltpu.sync_copy(sb_v, sampb_out)

        c1.wait()
        pltpu.sync_copy(b_v, trueb_out.at[pl.ds(base, _BPW)])

    return k(labels, nce_biases, sampled_ids)


_BB = 2048


def _select32(blk, ids):
    sel = lax.shift_right_logical(ids, 7) & 3
    out = jnp.zeros((blk.shape[0], DIM_), jnp.float32)
    for c in range(4):
        m = (sel == c).astype(jnp.float32)[:, None]
        out = out + m * blk[:, c * DIM_:(c + 1) * DIM_]
    return out


def _tc_loss_body(embblk_ref, wblk_ref, ti_ref, lb_ref, tb_ref,
                  sampblk_ref, sb_ref, sid_ref, embed_ref, out_ref):
    i = pl.program_id(0)
    ti = ti_ref[...]
    lb = lb_ref[...]
    sid = sid_ref[...]
    e = _select32(embblk_ref[...], ti)
    w = _select32(wblk_ref[...], lb)
    sw = _select32(sampblk_ref[...], sid)
    embed_ref[...] = e
    tl = jnp.sum(e * w, axis=1) + tb_ref[...]
    lf = lb.astype(jnp.float32)
    p_true = (jnp.log(lf + 2.0) - jnp.log(lf + 1.0)) / _LOG_VP1
    tl = tl - jnp.log(S_ * p_true)
    sl = lax.dot_general(e, sw, (((1,), (1,)), ((), ())),
                         preferred_element_type=jnp.float32)
    sf = sid.astype(jnp.float32)
    p_s = (jnp.log(sf + 2.0) - jnp.log(sf + 1.0)) / _LOG_VP1
    sl = sl + (sb_ref[...] - jnp.log(S_ * p_s))[None, :]
    ce_t = jnp.maximum(tl, 0.0) - tl + jnp.log1p(jnp.exp(-jnp.abs(tl)))
    ce_s = jnp.maximum(sl, 0.0) + jnp.log1p(jnp.exp(-jnp.abs(sl)))
    part = (jnp.sum(ce_t) + jnp.sum(ce_s)) * (1.0 / B_)

    @pl.when(i == 0)
    def _():
        out_ref[0, 0] = 0.0

    out_ref[0, 0] += part


def _tc_loss(embblk, wblk, train_inputs, labels, true_b, sampblk, sampled_b,
             sampled_ids, interpret=False):
    nblk = B_ // _BB
    embed, cost = pl.pallas_call(
        _tc_loss_body,
        grid=(nblk,),
        in_specs=[
            pl.BlockSpec((_BB, 128), lambda i: (i, 0)),
            pl.BlockSpec((_BB, 128), lambda i: (i, 0)),
            pl.BlockSpec((_BB,), lambda i: (i,)),
            pl.BlockSpec((_BB,), lambda i: (i,)),
            pl.BlockSpec((_BB,), lambda i: (i,)),
            pl.BlockSpec((S_, 128), lambda i: (0, 0)),
            pl.BlockSpec((S_,), lambda i: (0,)),
            pl.BlockSpec((S_,), lambda i: (0,)),
        ],
        out_specs=(
            pl.BlockSpec((_BB, DIM_), lambda i: (i, 0)),
            pl.BlockSpec((1, 1), lambda i: (0, 0), memory_space=pltpu.SMEM),
        ),
        out_shape=(
            jax.ShapeDtypeStruct((B_, DIM_), jnp.float32),
            jax.ShapeDtypeStruct((1, 1), jnp.float32),
        ),
        interpret=interpret,
    )(embblk, wblk, train_inputs, labels, true_b, sampblk, sampled_b,
      sampled_ids)
    return embed, cost[0, 0]


def kernel(train_inputs, train_labels, embeddings, nce_weights, nce_biases,
           sampled_ids):
    labels = train_labels.reshape(-1)
    emb_p = _tc_repack(embeddings.T)
    ncw_p = _tc_repack(nce_weights.T)
    embblk, wblk, sampblk = _sc_gather_rows(
        train_inputs, labels, emb_p, ncw_p, sampled_ids)
    true_b, sampled_b = _sc_gather_biases(labels, nce_biases, sampled_ids)
    embed, nce_cost = _tc_loss(embblk, wblk, train_inputs, labels, true_b,
                               sampblk, sampled_b, sampled_ids)
    return embed, nce_cost
